# baseline (speedup 1.0000x reference)
import jax
import jax.numpy as jnp
from jax import lax
from jax.experimental import pallas as pl
from jax.experimental.pallas import tpu as pltpu
from jax.experimental.pallas import tpu_sc as plsc

_B, _T, _C = 4, 2048, 1024
_BSC = 1
_BTC = _B - _BSC
_NC, _NS = 2, 16
_NW = _NC * _NS
_RPW = _T // _NW
_CH = 16
_STEPS = _RPW // _CH
_NIN = 3
_NOUT = 2
_NPE = 2
_NCHUNK = _STEPS * _BSC
_BT = 2048


def _sc_body(x_hbm, pe_hbm, out_hbm, xin, xout, pev,
             ldsems, stsems, pesems):
    wid = lax.axis_index("s") * _NC + lax.axis_index("c")
    rbase = wid * _RPW

    ld_desc = {}
    st_desc = {}
    pe_desc = {}

    def rows(k):
        s, b = divmod(k, _BSC)
        return b, pl.ds(rbase + s * _CH, _CH)

    def issue_load(k):
        buf = k % _NIN
        b, sl = rows(k)
        ld_desc[buf] = pltpu.async_copy(x_hbm.at[b, sl], xin[buf],
                                        ldsems[buf])

    def issue_pe(k):
        buf = k % _NPE
        _, sl = rows(k)
        pe_desc[buf] = pltpu.async_copy(pe_hbm.at[sl], pev[buf],
                                        pesems[buf])

    for k in range(min(_NIN, _NCHUNK)):
        issue_load(k)
    for k in range(min(_NPE, _NCHUNK)):
        issue_pe(k)

    for k in range(_NCHUNK):
        ibuf = k % _NIN
        obuf = k % _NOUT
        pbuf = k % _NPE
        ld_desc[ibuf].wait()
        pe_desc[pbuf].wait()
        if k >= _NOUT:
            st_desc[obuf].wait()
        xi = xin[ibuf]
        xo = xout[obuf]
        pv = pev[pbuf]

        @plsc.parallel_loop(0, _C, step=16, unroll=2)
        def _(j):
            sl = pl.ds(j, 16)
            for r in range(_CH):
                xo[r, sl] = xi[r, sl] + pv[r, sl]

        b, osl = rows(k)
        st_desc[obuf] = pltpu.async_copy(xo, out_hbm.at[b, osl],
                                         stsems[obuf])
        if k + _NIN < _NCHUNK:
            issue_load(k + _NIN)
        if k + _NPE < _NCHUNK:
            issue_pe(k + _NPE)

    for k in range(max(0, _NCHUNK - _NOUT), _NCHUNK):
        st_desc[k % _NOUT].wait()


def _sc_part(x, pos_emb):
    mesh = plsc.VectorSubcoreMesh(core_axis_name="c", subcore_axis_name="s")
    f = pl.kernel(
        _sc_body,
        mesh=mesh,
        out_type=jax.ShapeDtypeStruct((_BSC, _T, _C), jnp.float32),
        scratch_types=[
            [pltpu.VMEM((_CH, _C), jnp.float32)] * _NIN,
            [pltpu.VMEM((_CH, _C), jnp.float32)] * _NOUT,
            [pltpu.VMEM((_CH, _C), jnp.float32)] * _NPE,
            [pltpu.SemaphoreType.DMA] * _NIN,
            [pltpu.SemaphoreType.DMA] * _NOUT,
            [pltpu.SemaphoreType.DMA] * _NPE,
        ],
    )
    return f(x, pos_emb)


def _tc_add_body(x_ref, pe_ref, o_ref):
    o_ref[...] = x_ref[...] + pe_ref[...][None]


def _tc_part(x, pos_emb):
    return pl.pallas_call(
        _tc_add_body,
        grid=(_T // _BT, _BTC),
        in_specs=[
            pl.BlockSpec((1, _BT, _C), lambda t, b: (b + _BSC, t, 0)),
            pl.BlockSpec((_BT, _C), lambda t, b: (t, 0)),
        ],
        out_specs=pl.BlockSpec((1, _BT, _C), lambda t, b: (b + _BSC, t, 0)),
        out_shape=jax.ShapeDtypeStruct((_B, _T, _C), jnp.float32),
    )(x, pos_emb)


def kernel(x, pos_emb):
    tc_out = _tc_part(x, pos_emb)
    sc_out = _sc_part(x, pos_emb)
    sc_out = lax.optimization_barrier(sc_out)
    return lax.dynamic_update_slice(tc_out, sc_out, (0, 0, 0))


# speedup vs baseline: 1.0278x; 1.0278x over previous
import jax
import jax.numpy as jnp
from jax import lax
from jax.experimental import pallas as pl
from jax.experimental.pallas import tpu as pltpu
from jax.experimental.pallas import tpu_sc as plsc

_B, _T, _C = 4, 2048, 1024
_BSC = 1
_BTC = _B - _BSC
_NC, _NS = 2, 16
_NW = _NC * _NS
_RPW = _T // _NW
_CH = 8
_STEPS = _RPW // _CH
_NIN = 4
_NOUT = 4
_NPE = 4
_NCHUNK = _STEPS * _BSC
_BT = 2048


def _sc_body(x_hbm, pe_hbm, out_hbm, xin, xout, pev,
             ldsems, stsems, pesems):
    wid = lax.axis_index("s") * _NC + lax.axis_index("c")
    rbase = wid * _RPW

    ld_desc = {}
    st_desc = {}
    pe_desc = {}

    def rows(k):
        s, b = divmod(k, _BSC)
        return b, pl.ds(rbase + s * _CH, _CH)

    def issue_load(k):
        buf = k % _NIN
        b, sl = rows(k)
        ld_desc[buf] = pltpu.async_copy(x_hbm.at[b, sl], xin[buf],
                                        ldsems[buf])

    def issue_pe(k):
        buf = k % _NPE
        _, sl = rows(k)
        pe_desc[buf] = pltpu.async_copy(pe_hbm.at[sl], pev[buf],
                                        pesems[buf])

    for k in range(min(_NIN, _NCHUNK)):
        issue_load(k)
    for k in range(min(_NPE, _NCHUNK)):
        issue_pe(k)

    for k in range(_NCHUNK):
        ibuf = k % _NIN
        obuf = k % _NOUT
        pbuf = k % _NPE
        ld_desc[ibuf].wait()
        pe_desc[pbuf].wait()
        if k >= _NOUT:
            st_desc[obuf].wait()
        xi = xin[ibuf]
        xo = xout[obuf]
        pv = pev[pbuf]

        @plsc.parallel_loop(0, _C, step=16, unroll=2)
        def _(j):
            sl = pl.ds(j, 16)
            for r in range(_CH):
                xo[r, sl] = xi[r, sl] + pv[r, sl]

        b, osl = rows(k)
        st_desc[obuf] = pltpu.async_copy(xo, out_hbm.at[b, osl],
                                         stsems[obuf])
        if k + _NIN < _NCHUNK:
            issue_load(k + _NIN)
        if k + _NPE < _NCHUNK:
            issue_pe(k + _NPE)

    for k in range(max(0, _NCHUNK - _NOUT), _NCHUNK):
        st_desc[k % _NOUT].wait()


def _sc_part(x, pos_emb):
    mesh = plsc.VectorSubcoreMesh(core_axis_name="c", subcore_axis_name="s")
    f = pl.kernel(
        _sc_body,
        mesh=mesh,
        out_type=jax.ShapeDtypeStruct((_BSC, _T, _C), jnp.float32),
        scratch_types=[
            [pltpu.VMEM((_CH, _C), jnp.float32)] * _NIN,
            [pltpu.VMEM((_CH, _C), jnp.float32)] * _NOUT,
            [pltpu.VMEM((_CH, _C), jnp.float32)] * _NPE,
            [pltpu.SemaphoreType.DMA] * _NIN,
            [pltpu.SemaphoreType.DMA] * _NOUT,
            [pltpu.SemaphoreType.DMA] * _NPE,
        ],
    )
    return f(x, pos_emb)


def _tc_add_body(x_ref, pe_ref, o_ref):
    o_ref[...] = x_ref[...] + pe_ref[...][None]


def _tc_part(x, pos_emb):
    return pl.pallas_call(
        _tc_add_body,
        grid=(_T // _BT, _BTC),
        in_specs=[
            pl.BlockSpec((1, _BT, _C), lambda t, b: (b + _BSC, t, 0)),
            pl.BlockSpec((_BT, _C), lambda t, b: (t, 0)),
        ],
        out_specs=pl.BlockSpec((1, _BT, _C), lambda t, b: (b + _BSC, t, 0)),
        out_shape=jax.ShapeDtypeStruct((_B, _T, _C), jnp.float32),
    )(x, pos_emb)


def kernel(x, pos_emb):
    tc_out = _tc_part(x, pos_emb)
    sc_out = _sc_part(x, pos_emb)
    sc_out = lax.optimization_barrier(sc_out)
    return lax.dynamic_update_slice(tc_out, sc_out, (0, 0, 0))


# speedup vs baseline: 1.0301x; 1.0022x over previous
import jax
import jax.numpy as jnp
from jax import lax
from jax.experimental import pallas as pl
from jax.experimental.pallas import tpu as pltpu
from jax.experimental.pallas import tpu_sc as plsc

_B, _T, _C = 4, 2048, 1024
_BSC = 1
_BTC = _B - _BSC
_NC, _NS = 2, 16
_NW = _NC * _NS
_RPW = _T // _NW
_CH = 8
_STEPS = _RPW // _CH
_NIN = 5
_NOUT = 5
_NPE = 5
_NCHUNK = _STEPS * _BSC
_BT = 2048


def _sc_body(x_hbm, pe_hbm, out_hbm, xin, xout, pev,
             ldsems, stsems, pesems):
    wid = lax.axis_index("s") * _NC + lax.axis_index("c")
    rbase = wid * _RPW

    ld_desc = {}
    st_desc = {}
    pe_desc = {}

    def rows(k):
        s, b = divmod(k, _BSC)
        return b, pl.ds(rbase + s * _CH, _CH)

    def issue_load(k):
        buf = k % _NIN
        b, sl = rows(k)
        ld_desc[buf] = pltpu.async_copy(x_hbm.at[b, sl], xin[buf],
                                        ldsems[buf])

    def issue_pe(k):
        buf = k % _NPE
        _, sl = rows(k)
        pe_desc[buf] = pltpu.async_copy(pe_hbm.at[sl], pev[buf],
                                        pesems[buf])

    for k in range(min(_NIN, _NCHUNK)):
        issue_load(k)
    for k in range(min(_NPE, _NCHUNK)):
        issue_pe(k)

    for k in range(_NCHUNK):
        ibuf = k % _NIN
        obuf = k % _NOUT
        pbuf = k % _NPE
        ld_desc[ibuf].wait()
        pe_desc[pbuf].wait()
        if k >= _NOUT:
            st_desc[obuf].wait()
        xi = xin[ibuf]
        xo = xout[obuf]
        pv = pev[pbuf]

        @plsc.parallel_loop(0, _C, step=16, unroll=2)
        def _(j):
            sl = pl.ds(j, 16)
            for r in range(_CH):
                xo[r, sl] = xi[r, sl] + pv[r, sl]

        b, osl = rows(k)
        st_desc[obuf] = pltpu.async_copy(xo, out_hbm.at[b, osl],
                                         stsems[obuf])
        if k + _NIN < _NCHUNK:
            issue_load(k + _NIN)
        if k + _NPE < _NCHUNK:
            issue_pe(k + _NPE)

    for k in range(max(0, _NCHUNK - _NOUT), _NCHUNK):
        st_desc[k % _NOUT].wait()


def _sc_part(x, pos_emb):
    mesh = plsc.VectorSubcoreMesh(core_axis_name="c", subcore_axis_name="s")
    f = pl.kernel(
        _sc_body,
        mesh=mesh,
        out_type=jax.ShapeDtypeStruct((_BSC, _T, _C), jnp.float32),
        scratch_types=[
            [pltpu.VMEM((_CH, _C), jnp.float32)] * _NIN,
            [pltpu.VMEM((_CH, _C), jnp.float32)] * _NOUT,
            [pltpu.VMEM((_CH, _C), jnp.float32)] * _NPE,
            [pltpu.SemaphoreType.DMA] * _NIN,
            [pltpu.SemaphoreType.DMA] * _NOUT,
            [pltpu.SemaphoreType.DMA] * _NPE,
        ],
    )
    return f(x, pos_emb)


def _tc_add_body(x_ref, pe_ref, o_ref):
    o_ref[...] = x_ref[...] + pe_ref[...][None]


def _tc_part(x, pos_emb):
    return pl.pallas_call(
        _tc_add_body,
        grid=(_T // _BT, _BTC),
        in_specs=[
            pl.BlockSpec((1, _BT, _C), lambda t, b: (b + _BSC, t, 0)),
            pl.BlockSpec((_BT, _C), lambda t, b: (t, 0)),
        ],
        out_specs=pl.BlockSpec((1, _BT, _C), lambda t, b: (b + _BSC, t, 0)),
        out_shape=jax.ShapeDtypeStruct((_B, _T, _C), jnp.float32),
    )(x, pos_emb)


def kernel(x, pos_emb):
    tc_out = _tc_part(x, pos_emb)
    sc_out = _sc_part(x, pos_emb)
    sc_out = lax.optimization_barrier(sc_out)
    return lax.dynamic_update_slice(tc_out, sc_out, (0, 0, 0))
